# Initial kernel scaffold; baseline (speedup 1.0000x reference)
import jax
import jax.numpy as jnp
from jax import lax
from jax.experimental import pallas as pl
from jax.experimental.pallas import tpu as pltpu
from jax.experimental.pallas import tpu_sc as plsc

NUM_EMBEDDINGS = 1000000
EMBEDDING_DIM = 32
BATCH = 4096
SEQ_LEN = 200

_NW = 32
_B = BATCH * SEQ_LEN
_PER_W = _B // _NW
_CHUNK = 1600
_NCHUNK = _PER_W // _CHUNK


def _body(idx_hbm, table_hbm, out_hbm, idx_v, rows_v, sem):
    cid = lax.axis_index("c")
    sid = lax.axis_index("s")
    wid = sid * 2 + cid
    w_base = wid * _PER_W

    def chunk(j, carry):
        base = w_base + j * _CHUNK
        pltpu.sync_copy(idx_hbm.at[pl.ds(base, _CHUNK)], idx_v)
        pltpu.async_copy(table_hbm.at[idx_v], rows_v, sem).wait()
        pltpu.sync_copy(rows_v, out_hbm.at[pl.ds(base, _CHUNK)])
        return carry

    lax.fori_loop(0, _NCHUNK, chunk, 0)


@jax.jit
def kernel(token_ids, embedding):
    flat_ids = token_ids.reshape(_B).astype(jnp.int32)
    mesh = plsc.VectorSubcoreMesh(core_axis_name="c", subcore_axis_name="s")
    out = pl.kernel(
        _body,
        out_type=jax.ShapeDtypeStruct((_B, EMBEDDING_DIM), jnp.float32),
        mesh=mesh,
        scratch_types=[
            pltpu.VMEM((_CHUNK,), jnp.int32),
            pltpu.VMEM((_CHUNK, EMBEDDING_DIM), jnp.float32),
            pltpu.SemaphoreType.DMA,
        ],
    )(flat_ids, embedding)
    return out.reshape(BATCH, SEQ_LEN, EMBEDDING_DIM)


# speedup vs baseline: 1.4782x; 1.4782x over previous
import jax
import jax.numpy as jnp
from jax import lax
from jax.experimental import pallas as pl
from jax.experimental.pallas import tpu as pltpu
from jax.experimental.pallas import tpu_sc as plsc

NUM_EMBEDDINGS = 1000000
EMBEDDING_DIM = 32
BATCH = 4096
SEQ_LEN = 200

_NW = 32
_B = BATCH * SEQ_LEN
_PER_W = _B // _NW
_CHUNK = 1600
_NCHUNK = _PER_W // _CHUNK


def _body(idx_hbm, table_hbm, out_hbm, idx_v, rows_v, sem):
    cid = lax.axis_index("c")
    sid = lax.axis_index("s")
    wid = sid * 2 + cid
    w_base = wid * _PER_W

    def chunk(j, carry):
        base = w_base + j * _CHUNK
        pltpu.sync_copy(idx_hbm.at[pl.ds(base, _CHUNK)], idx_v)
        pltpu.async_copy(table_hbm.at[idx_v], rows_v, sem).wait()
        pltpu.sync_copy(rows_v, out_hbm.at[pl.ds(base, _CHUNK)])
        return carry

    lax.fori_loop(0, _NCHUNK, chunk, 0)


@jax.jit
def kernel(token_ids, embedding):
    flat_ids = token_ids.reshape(_B).astype(jnp.int32)
    mesh = plsc.VectorSubcoreMesh(core_axis_name="c", subcore_axis_name="s")
    out = pl.kernel(
        _body,
        out_type=jax.ShapeDtypeStruct((_B, EMBEDDING_DIM), jnp.float32),
        mesh=mesh,
        scratch_types=[
            pltpu.VMEM((_CHUNK,), jnp.int32),
            pltpu.VMEM((_CHUNK, EMBEDDING_DIM), jnp.float32),
            pltpu.SemaphoreType.DMA,
        ],
        compiler_params=pltpu.CompilerParams(use_tc_tiling_on_sc=False),
    )(flat_ids, embedding)
    return out.reshape(BATCH, SEQ_LEN, EMBEDDING_DIM)


# speedup vs baseline: 1.5017x; 1.0159x over previous
import jax
import jax.numpy as jnp
from jax import lax
from jax.experimental import pallas as pl
from jax.experimental.pallas import tpu as pltpu
from jax.experimental.pallas import tpu_sc as plsc

NUM_EMBEDDINGS = 1000000
EMBEDDING_DIM = 32
BATCH = 4096
SEQ_LEN = 200

_NW = 32
_B = BATCH * SEQ_LEN
_PER_W = _B // _NW
_CHUNK = 1600
_NCHUNK = _PER_W // _CHUNK


def _body(idx_hbm, table_hbm, out_hbm, idx_all, rows0, rows1, g0, g1, w0, w1):
    cid = lax.axis_index("c")
    sid = lax.axis_index("s")
    wid = sid * 2 + cid
    w_base = wid * _PER_W

    pltpu.sync_copy(idx_hbm.at[wid], idx_all)

    rows = (rows0, rows1)
    gsem = (g0, g1)
    wsem = (w0, w1)
    gdesc = [None] * _NCHUNK
    wdesc = [None] * _NCHUNK
    gdesc[0] = pltpu.async_copy(table_hbm.at[idx_all.at[0]], rows[0], gsem[0])
    gdesc[1] = pltpu.async_copy(table_hbm.at[idx_all.at[1]], rows[1], gsem[1])
    for j in range(_NCHUNK):
        b = j % 2
        gdesc[j].wait()
        wdesc[j] = pltpu.async_copy(
            rows[b], out_hbm.at[pl.ds(w_base + j * _CHUNK, _CHUNK)], wsem[b]
        )
        if j + 2 < _NCHUNK:
            wdesc[j].wait()
            gdesc[j + 2] = pltpu.async_copy(
                table_hbm.at[idx_all.at[j + 2]], rows[b], gsem[b]
            )
    wdesc[_NCHUNK - 2].wait()
    wdesc[_NCHUNK - 1].wait()


@jax.jit
def kernel(token_ids, embedding):
    flat_ids = token_ids.reshape(_NW, _NCHUNK, _CHUNK).astype(jnp.int32)
    mesh = plsc.VectorSubcoreMesh(core_axis_name="c", subcore_axis_name="s")
    out = pl.kernel(
        _body,
        out_type=jax.ShapeDtypeStruct((_B, EMBEDDING_DIM), jnp.float32),
        mesh=mesh,
        scratch_types=[
            pltpu.VMEM((_NCHUNK, _CHUNK), jnp.int32),
            pltpu.VMEM((_CHUNK, EMBEDDING_DIM), jnp.float32),
            pltpu.VMEM((_CHUNK, EMBEDDING_DIM), jnp.float32),
            pltpu.SemaphoreType.DMA,
            pltpu.SemaphoreType.DMA,
            pltpu.SemaphoreType.DMA,
            pltpu.SemaphoreType.DMA,
        ],
        compiler_params=pltpu.CompilerParams(use_tc_tiling_on_sc=False),
    )(flat_ids, embedding)
    return out.reshape(BATCH, SEQ_LEN, EMBEDDING_DIM)
